# baseline (speedup 1.0000x reference)
import functools

import jax
import jax.numpy as jnp
from jax import lax
from jax.experimental import pallas as pl
from jax.experimental.pallas import tpu as pltpu
from jax.experimental.pallas import tpu_sc as plsc

B, S, D = 1, 2048, 1024
E, K = 16, 2
DFF = 704
DFF_SH = 1408

T = B * S
TK = T * K
TM = 256
NT = E + TK // TM
NR = NT * TM

NC, NS = 2, 16
NW = NC * NS

LCH = 128
NCH = TK // LCH

TT = 512
NTT = T // TT


def _gate_kernel(x_ref, gw_ref, w_ref, slot_ref, tbl_ref):
    x = x_ref[...]
    logits = jnp.dot(x, gw_ref[...].T, preferred_element_type=jnp.float32)
    m = jnp.max(logits, axis=-1, keepdims=True)
    ex = jnp.exp(logits - m)
    scores = ex / jnp.sum(ex, axis=-1, keepdims=True)

    iota = lax.broadcasted_iota(jnp.int32, scores.shape, 1)
    v1 = jnp.max(scores, axis=-1, keepdims=True)
    i1 = jnp.min(jnp.where(scores == v1, iota, E), axis=-1, keepdims=True)
    masked = jnp.where(iota == i1, -jnp.inf, scores)
    v2 = jnp.max(masked, axis=-1, keepdims=True)
    i2 = jnp.min(jnp.where(masked == v2, iota, E), axis=-1, keepdims=True)

    zf = jnp.zeros((T, 126), jnp.float32)
    w_ref[...] = jnp.concatenate([v1, v2, zf], axis=1)

    A = jnp.concatenate(
        [i1[:, 0].reshape(LCH, 16), i2[:, 0].reshape(LCH, 16)], axis=1
    ).astype(jnp.float32)

    CE_ = NCH * E
    rep = (lax.broadcasted_iota(jnp.int32, (NCH, CE_), 0)
           == lax.broadcasted_iota(jnp.int32, (NCH, CE_), 1) // E
           ).astype(jnp.float32)
    Aexp = jnp.dot(A, rep, preferred_element_type=jnp.float32)
    epat = (lax.broadcasted_iota(jnp.int32, (LCH, CE_), 1) % E
            ).astype(jnp.float32)
    X = (Aexp == epat).astype(jnp.float32)

    li = lax.broadcasted_iota(jnp.int32, (LCH, LCH), 0)
    lj = lax.broadcasted_iota(jnp.int32, (LCH, LCH), 1)
    tril = (lj < li).astype(jnp.float32)
    Sx = jnp.dot(tril, X, preferred_element_type=jnp.float32)
    tot = jnp.sum(X, axis=0, keepdims=True)

    cs_re = lax.broadcasted_iota(jnp.int32, (CE_, CE_), 0)
    cs_co = lax.broadcasted_iota(jnp.int32, (CE_, CE_), 1)
    csm = ((cs_re % E == cs_co % E) & (cs_re // E < cs_co // E)
           ).astype(jnp.float32)
    coff = jnp.dot(tot, csm, preferred_element_type=jnp.float32)

    cem = ((lax.broadcasted_iota(jnp.int32, (CE_, E), 0) % E)
           == lax.broadcasted_iota(jnp.int32, (CE_, E), 1)
           ).astype(jnp.float32)
    counts = jnp.dot(tot, cem, preferred_element_type=jnp.float32)

    ei = lax.broadcasted_iota(jnp.int32, (E, E), 0)
    ej = lax.broadcasted_iota(jnp.int32, (E, E), 1)
    tril_e = (ei < ej).astype(jnp.float32)
    tpe = jnp.floor((counts + (TM - 1)) * (1.0 / TM))
    ft = jnp.dot(tpe, tril_e, preferred_element_type=jnp.float32)

    fem = (lax.broadcasted_iota(jnp.int32, (E, CE_), 0)
           == lax.broadcasted_iota(jnp.int32, (E, CE_), 1) % E
           ).astype(jnp.float32)
    ftm = jnp.dot(ft, fem, preferred_element_type=jnp.float32)

    r = coff + Sx
    rq = jnp.floor(r * (1.0 / TM))
    rr = r - rq * TM
    ccm = ((lax.broadcasted_iota(jnp.int32, (CE_, NCH), 0) // E)
           == lax.broadcasted_iota(jnp.int32, (CE_, NCH), 1)
           ).astype(jnp.float32)
    tile_part = jnp.dot(X * (ftm + rq), ccm,
                        preferred_element_type=jnp.float32)
    rem_part = jnp.dot(X * rr, ccm,
                       preferred_element_type=jnp.float32)
    slot = tile_part * TM + rem_part
    slot_ref[...] = slot.astype(jnp.int32)

    ti = lax.broadcasted_iota(jnp.int32, (NT, E), 0).astype(jnp.float32)
    ftb = jnp.broadcast_to(ft, (NT, E))
    te = jnp.sum((ti >= ftb).astype(jnp.float32), axis=1, keepdims=True) - 1.0
    teo = (lax.broadcasted_iota(jnp.int32, (NT, E), 1).astype(jnp.float32)
           == te).astype(jnp.float32)
    cnt_te = jnp.sum(teo * counts, axis=1, keepdims=True)
    ft_te = jnp.sum(teo * ftb, axis=1, keepdims=True)
    within = ti[:, :1] - ft_te
    nrows = jnp.clip(cnt_te - within * TM, 0.0, TM)
    pad = jnp.zeros((NT, 126), jnp.float32)
    tbl_ref[...] = jnp.concatenate([te, nrows, pad], axis=1).astype(jnp.int32)


def _gate(x, gate_weight):
    return pl.pallas_call(
        _gate_kernel,
        out_shape=(
            jax.ShapeDtypeStruct((T, 128), jnp.float32),
            jax.ShapeDtypeStruct((LCH, NCH), jnp.int32),
            jax.ShapeDtypeStruct((NT, 128), jnp.int32),
        ),
    )(x, gate_weight)


def _make_sc_gather(nrows, ch, width):
    rpw = nrows // NW
    mesh = plsc.VectorSubcoreMesh(core_axis_name="c", subcore_axis_name="s")

    @functools.partial(
        pl.kernel,
        mesh=mesh,
        out_type=jax.ShapeDtypeStruct((nrows, width), jnp.int32),
        scratch_types=[
            pltpu.VMEM((ch,), jnp.int32),
            pltpu.VMEM((ch, width), jnp.int32),
            pltpu.VMEM((ch, width), jnp.int32),
            pltpu.SemaphoreType.DMA,
            pltpu.SemaphoreType.DMA,
        ],
    )
    def k(table_hbm, idx_hbm, out_hbm, idx_v, rows0, rows1, sem0, sem1):
        wid = lax.axis_index("s") * NC + lax.axis_index("c")
        nchunk = rpw // ch
        bufs = (rows0, rows1)
        sems = (sem0, sem1)
        base0 = wid * rpw
        pltpu.sync_copy(idx_hbm.at[pl.ds(base0, ch)], idx_v)
        cp = pltpu.async_copy(table_hbm.at[idx_v], rows0, sem0)
        for c in range(nchunk):
            cp.wait()
            if c + 1 < nchunk:
                pltpu.sync_copy(idx_hbm.at[pl.ds(base0 + (c + 1) * ch, ch)],
                                idx_v)
                cp = pltpu.async_copy(table_hbm.at[idx_v],
                                     bufs[(c + 1) % 2], sems[(c + 1) % 2])
            pltpu.sync_copy(bufs[c % 2], out_hbm.at[pl.ds(base0 + c * ch, ch)])

    return k


def _gather_rows(table, idx, nrows, ch):
    if table.dtype == jnp.bfloat16:
        w2 = table.shape[1] // 2
        ti = lax.bitcast_convert_type(table.reshape(-1, w2, 2), jnp.int32)
        out = _make_sc_gather(nrows, ch, w2)(ti, idx)
        return lax.bitcast_convert_type(out, jnp.bfloat16).reshape(nrows, -1)
    ti = lax.bitcast_convert_type(table, jnp.int32)
    out = _make_sc_gather(nrows, ch, table.shape[1])(ti, idx)
    return lax.bitcast_convert_type(out, table.dtype)


def _gemm_kernel(te_ref, nr_ref, xs_ref, w_ref, guw_ref, dw_ref, out_ref):
    i = pl.program_id(0)

    @pl.when(nr_ref[i] > 0)
    def _():
        xb = xs_ref[...]
        guw = guw_ref[0].astype(jnp.bfloat16)
        gu = jnp.dot(xb, guw.T, preferred_element_type=jnp.float32)
        g = gu[:, :DFF]
        u = gu[:, DFF:]
        h = (g * jax.nn.sigmoid(g)) * u
        out = jnp.dot(h.astype(jnp.bfloat16), dw_ref[0].astype(jnp.bfloat16).T,
                      preferred_element_type=jnp.float32)
        out_ref[...] = (out * w_ref[0, 0, :][:, None]).astype(jnp.bfloat16)


def _grouped_gemm(te, nrows, xs, w_pad, guw, dw):
    grid_spec = pltpu.PrefetchScalarGridSpec(
        num_scalar_prefetch=2,
        grid=(NT,),
        in_specs=[
            pl.BlockSpec((TM, D), lambda i, te, nr: (i, 0)),
            pl.BlockSpec((1, 1, TM), lambda i, te, nr: (i, 0, 0)),
            pl.BlockSpec((1, 2 * DFF, D), lambda i, te, nr: (te[i], 0, 0)),
            pl.BlockSpec((1, D, DFF), lambda i, te, nr: (te[i], 0, 0)),
        ],
        out_specs=pl.BlockSpec((TM, D), lambda i, te, nr: (i, 0)),
    )
    return pl.pallas_call(
        _gemm_kernel,
        grid_spec=grid_spec,
        out_shape=jax.ShapeDtypeStruct((NR, D), jnp.bfloat16),
    )(te, nrows, xs, w_pad, guw, dw)


def _shared_kernel(x_ref, sguw_ref, sdw_ref, a_ref, b_ref, y_ref):
    xb = x_ref[...].astype(jnp.bfloat16)
    sguw = sguw_ref[...].astype(jnp.bfloat16)
    su = jnp.dot(xb, sguw.T, preferred_element_type=jnp.float32)
    sg = su[:, :DFF_SH]
    up = su[:, DFF_SH:]
    hs = (sg * jax.nn.sigmoid(sg)) * up
    y = jnp.dot(hs.astype(jnp.bfloat16), sdw_ref[...].astype(jnp.bfloat16).T,
                preferred_element_type=jnp.float32)
    y_ref[...] = y + a_ref[...].astype(jnp.float32) \
        + b_ref[...].astype(jnp.float32)


def _shared_combine(x, sguw, sdw, ab_rows):
    return pl.pallas_call(
        _shared_kernel,
        grid=(NTT,),
        in_specs=[
            pl.BlockSpec((TT, D), lambda i: (i, 0)),
            pl.BlockSpec((2 * DFF_SH, D), lambda i: (0, 0)),
            pl.BlockSpec((D, DFF_SH), lambda i: (0, 0)),
            pl.BlockSpec((TT, D), lambda i: (i, 0)),
            pl.BlockSpec((TT, D), lambda i: (T // TT + i, 0)),
        ],
        out_specs=pl.BlockSpec((TT, D), lambda i: (i, 0)),
        out_shape=jax.ShapeDtypeStruct((T, D), jnp.float32),
    )(x, sguw, sdw, ab_rows, ab_rows)


@jax.jit
def kernel(hidden_states, gate_weight, gate_up_weights, down_weights,
           shared_gate_w, shared_up_w, shared_down_w):
    x = hidden_states.reshape(-1, D)
    sguw = jnp.concatenate([shared_gate_w, shared_up_w], axis=0)

    wout, slotmat, tbl = _gate(x, gate_weight)
    te = tbl[:, 0]
    nrows = tbl[:, 1]
    c0 = slotmat[:, :16].reshape(T)
    c1 = slotmat[:, 16:].reshape(T)
    cidx = jnp.concatenate([c0, c1])
    tw = jnp.concatenate([wout[:, 0], wout[:, 1]])
    ar = jnp.arange(T, dtype=jnp.int32)
    toks = jnp.concatenate([ar, ar])

    pad_ids = jnp.arange(NR, dtype=jnp.int32) % T
    gidx = pad_ids.at[cidx].set(toks)
    w_pad = jnp.zeros((NR,), jnp.float32).at[cidx].set(tw).reshape(NT, 1, TM)

    xb16 = x.astype(jnp.bfloat16)
    xs = _gather_rows(xb16, gidx, NR, 64)
    out_rows = _grouped_gemm(te, nrows, xs, w_pad, gate_up_weights,
                             down_weights)
    ab_rows = _gather_rows(out_rows, cidx, 2 * T, 64)
    y = _shared_combine(x, sguw, sdw=shared_down_w, ab_rows=ab_rows)

    return y.reshape(B, S, D)


# speedup vs baseline: 2.4947x; 2.4947x over previous
import functools

import jax
import jax.numpy as jnp
from jax import lax
from jax.experimental import pallas as pl
from jax.experimental.pallas import tpu as pltpu
from jax.experimental.pallas import tpu_sc as plsc

B, S, D = 1, 2048, 1024
E, K = 16, 2
DFF = 704
DFF_SH = 1408

T = B * S
TK = T * K
TM = 256
NT = E + TK // TM
NR = NT * TM

NC, NS = 2, 16
NW = NC * NS

LCH = 128
NCH = TK // LCH

TT = 512
NTT = T // TT


def _gate_kernel(x_ref, gw_ref, w_ref, slot_ref, tbl_ref):
    x = x_ref[...]
    logits = jnp.dot(x, gw_ref[...].T, preferred_element_type=jnp.float32)
    m = jnp.max(logits, axis=-1, keepdims=True)
    ex = jnp.exp(logits - m)
    scores = ex / jnp.sum(ex, axis=-1, keepdims=True)

    iota = lax.broadcasted_iota(jnp.int32, scores.shape, 1)
    v1 = jnp.max(scores, axis=-1, keepdims=True)
    i1 = jnp.min(jnp.where(scores == v1, iota, E), axis=-1, keepdims=True)
    masked = jnp.where(iota == i1, -jnp.inf, scores)
    v2 = jnp.max(masked, axis=-1, keepdims=True)
    i2 = jnp.min(jnp.where(masked == v2, iota, E), axis=-1, keepdims=True)

    zf = jnp.zeros((T, 126), jnp.float32)
    w_ref[...] = jnp.concatenate([v1, v2, zf], axis=1)

    A = jnp.concatenate(
        [i1[:, 0].reshape(LCH, 16), i2[:, 0].reshape(LCH, 16)], axis=1
    ).astype(jnp.float32)

    CE_ = NCH * E
    rep = (lax.broadcasted_iota(jnp.int32, (NCH, CE_), 0)
           == lax.broadcasted_iota(jnp.int32, (NCH, CE_), 1) // E
           ).astype(jnp.float32)
    Aexp = jnp.dot(A, rep, preferred_element_type=jnp.float32)
    epat = (lax.broadcasted_iota(jnp.int32, (LCH, CE_), 1) % E
            ).astype(jnp.float32)
    X = (Aexp == epat).astype(jnp.float32)

    li = lax.broadcasted_iota(jnp.int32, (LCH, LCH), 0)
    lj = lax.broadcasted_iota(jnp.int32, (LCH, LCH), 1)
    tril = (lj < li).astype(jnp.float32)
    Sx = jnp.dot(tril, X, preferred_element_type=jnp.float32)
    tot = jnp.sum(X, axis=0, keepdims=True)

    cs_re = lax.broadcasted_iota(jnp.int32, (CE_, CE_), 0)
    cs_co = lax.broadcasted_iota(jnp.int32, (CE_, CE_), 1)
    csm = ((cs_re % E == cs_co % E) & (cs_re // E < cs_co // E)
           ).astype(jnp.float32)
    coff = jnp.dot(tot, csm, preferred_element_type=jnp.float32)

    cem = ((lax.broadcasted_iota(jnp.int32, (CE_, E), 0) % E)
           == lax.broadcasted_iota(jnp.int32, (CE_, E), 1)
           ).astype(jnp.float32)
    counts = jnp.dot(tot, cem, preferred_element_type=jnp.float32)

    ei = lax.broadcasted_iota(jnp.int32, (E, E), 0)
    ej = lax.broadcasted_iota(jnp.int32, (E, E), 1)
    tril_e = (ei < ej).astype(jnp.float32)
    tpe = jnp.floor((counts + (TM - 1)) * (1.0 / TM))
    ft = jnp.dot(tpe, tril_e, preferred_element_type=jnp.float32)

    fem = (lax.broadcasted_iota(jnp.int32, (E, CE_), 0)
           == lax.broadcasted_iota(jnp.int32, (E, CE_), 1) % E
           ).astype(jnp.float32)
    ftm = jnp.dot(ft, fem, preferred_element_type=jnp.float32)

    r = coff + Sx
    rq = jnp.floor(r * (1.0 / TM))
    rr = r - rq * TM
    ccm = ((lax.broadcasted_iota(jnp.int32, (CE_, NCH), 0) // E)
           == lax.broadcasted_iota(jnp.int32, (CE_, NCH), 1)
           ).astype(jnp.float32)
    tile_part = jnp.dot(X * (ftm + rq), ccm,
                        preferred_element_type=jnp.float32)
    rem_part = jnp.dot(X * rr, ccm,
                       preferred_element_type=jnp.float32)
    slot = tile_part * TM + rem_part
    slot_ref[...] = slot.astype(jnp.int32)

    ti = lax.broadcasted_iota(jnp.int32, (NT, E), 0).astype(jnp.float32)
    ftb = jnp.broadcast_to(ft, (NT, E))
    te = jnp.sum((ti >= ftb).astype(jnp.float32), axis=1, keepdims=True) - 1.0
    teo = (lax.broadcasted_iota(jnp.int32, (NT, E), 1).astype(jnp.float32)
           == te).astype(jnp.float32)
    cnt_te = jnp.sum(teo * counts, axis=1, keepdims=True)
    ft_te = jnp.sum(teo * ftb, axis=1, keepdims=True)
    within = ti[:, :1] - ft_te
    nrows = jnp.clip(cnt_te - within * TM, 0.0, TM)
    pad = jnp.zeros((NT, 126), jnp.float32)
    tbl_ref[...] = jnp.concatenate([te, nrows, pad], axis=1).astype(jnp.int32)


def _gate(x, gate_weight):
    return pl.pallas_call(
        _gate_kernel,
        out_shape=(
            jax.ShapeDtypeStruct((T, 128), jnp.float32),
            jax.ShapeDtypeStruct((LCH, NCH), jnp.int32),
            jax.ShapeDtypeStruct((NT, 128), jnp.int32),
        ),
    )(x, gate_weight)


def _make_sc_gather(nrows, ch):
    rpw = nrows // NW
    mesh = plsc.VectorSubcoreMesh(core_axis_name="c", subcore_axis_name="s")

    @functools.partial(
        pl.kernel,
        mesh=mesh,
        out_type=jax.ShapeDtypeStruct((nrows, D), jnp.float32),
        scratch_types=[
            pltpu.VMEM((ch,), jnp.int32),
            pltpu.VMEM((ch, D), jnp.float32),
            pltpu.SemaphoreType.DMA,
        ],
    )
    def k(table_hbm, idx_hbm, out_hbm, idx_v, rows_v, sem):
        wid = lax.axis_index("s") * NC + lax.axis_index("c")
        for c in range(rpw // ch):
            base = wid * rpw + c * ch
            pltpu.sync_copy(idx_hbm.at[pl.ds(base, ch)], idx_v)
            pltpu.async_copy(table_hbm.at[idx_v], rows_v, sem).wait()
            pltpu.sync_copy(rows_v, out_hbm.at[pl.ds(base, ch)])

    return k


def _gather_rows(table, idx, nrows, ch):
    return _make_sc_gather(nrows, ch)(table, idx)


def _gemm_kernel(te_ref, nr_ref, xs_ref, w_ref, guw_ref, dw_ref, out_ref):
    i = pl.program_id(0)

    @pl.when(nr_ref[i] > 0)
    def _():
        xb = xs_ref[...].astype(jnp.bfloat16)
        guw = guw_ref[0].astype(jnp.bfloat16)
        gu = jnp.dot(xb, guw.T, preferred_element_type=jnp.float32)
        g = gu[:, :DFF]
        u = gu[:, DFF:]
        h = (g * jax.nn.sigmoid(g)) * u
        out = jnp.dot(h.astype(jnp.bfloat16), dw_ref[0].astype(jnp.bfloat16).T,
                      preferred_element_type=jnp.float32)
        out_ref[...] = out * w_ref[0, 0, :][:, None]


def _grouped_gemm(te, nrows, xs, w_pad, guw, dw):
    grid_spec = pltpu.PrefetchScalarGridSpec(
        num_scalar_prefetch=2,
        grid=(NT,),
        in_specs=[
            pl.BlockSpec((TM, D), lambda i, te, nr: (i, 0)),
            pl.BlockSpec((1, 1, TM), lambda i, te, nr: (i, 0, 0)),
            pl.BlockSpec((1, 2 * DFF, D), lambda i, te, nr: (te[i], 0, 0)),
            pl.BlockSpec((1, D, DFF), lambda i, te, nr: (te[i], 0, 0)),
        ],
        out_specs=pl.BlockSpec((TM, D), lambda i, te, nr: (i, 0)),
    )
    return pl.pallas_call(
        _gemm_kernel,
        grid_spec=grid_spec,
        out_shape=jax.ShapeDtypeStruct((NR, D), jnp.float32),
    )(te, nrows, xs, w_pad, guw, dw)


def _shared_kernel(x_ref, sguw_ref, sdw_ref, a_ref, b_ref, y_ref):
    xb = x_ref[...].astype(jnp.bfloat16)
    sguw = sguw_ref[...].astype(jnp.bfloat16)
    su = jnp.dot(xb, sguw.T, preferred_element_type=jnp.float32)
    sg = su[:, :DFF_SH]
    up = su[:, DFF_SH:]
    hs = (sg * jax.nn.sigmoid(sg)) * up
    y = jnp.dot(hs.astype(jnp.bfloat16), sdw_ref[...].astype(jnp.bfloat16).T,
                preferred_element_type=jnp.float32)
    y_ref[...] = y + a_ref[...].astype(jnp.float32) \
        + b_ref[...].astype(jnp.float32)


def _shared_combine(x, sguw, sdw, ab_rows):
    return pl.pallas_call(
        _shared_kernel,
        grid=(NTT,),
        in_specs=[
            pl.BlockSpec((TT, D), lambda i: (i, 0)),
            pl.BlockSpec((2 * DFF_SH, D), lambda i: (0, 0)),
            pl.BlockSpec((D, DFF_SH), lambda i: (0, 0)),
            pl.BlockSpec((TT, D), lambda i: (i, 0)),
            pl.BlockSpec((TT, D), lambda i: (T // TT + i, 0)),
        ],
        out_specs=pl.BlockSpec((TT, D), lambda i: (i, 0)),
        out_shape=jax.ShapeDtypeStruct((T, D), jnp.float32),
    )(x, sguw, sdw, ab_rows, ab_rows)


@jax.jit
def kernel(hidden_states, gate_weight, gate_up_weights, down_weights,
           shared_gate_w, shared_up_w, shared_down_w):
    x = hidden_states.reshape(-1, D)
    sguw = jnp.concatenate([shared_gate_w, shared_up_w], axis=0)

    wout, slotmat, tbl = _gate(x, gate_weight)
    te = tbl[:, 0]
    nrows = tbl[:, 1]
    c0 = slotmat[:, :16].reshape(T)
    c1 = slotmat[:, 16:].reshape(T)
    cidx = jnp.concatenate([c0, c1])
    tw = jnp.concatenate([wout[:, 0], wout[:, 1]])
    ar = jnp.arange(T, dtype=jnp.int32)
    toks = jnp.concatenate([ar, ar])

    pad_ids = jnp.arange(NR, dtype=jnp.int32) % T
    gidx = pad_ids.at[cidx].set(toks)
    w_pad = jnp.zeros((NR,), jnp.float32).at[cidx].set(tw).reshape(NT, 1, TM)

    xs = _gather_rows(x, gidx, NR, 64)
    out_rows = _grouped_gemm(te, nrows, xs, w_pad, gate_up_weights,
                             down_weights)
    ab_rows = _gather_rows(out_rows, cidx, 2 * T, 64)
    y = _shared_combine(x, sguw, sdw=shared_down_w, ab_rows=ab_rows)

    return y.reshape(B, S, D)
